# baseline (speedup 1.0000x reference)
import jax
import jax.numpy as jnp
from jax import lax
from jax.experimental import pallas as pl
from jax.experimental.pallas import tpu as pltpu
from jax.experimental.pallas import tpu_sc as plsc

NC = 2
NS = 16
NW = NC * NS
CHUNK = 128
CPG = 2
GROUP = CHUNK * CPG
NSET = 3


def _gather_body(table_hbm, idx_hbm, out_hbm, idx_v,
                 rows0, rows1, rows2, g0, g1, g2, w0, w1, w2):
    rows = (rows0, rows1, rows2)
    gsem = (g0, g1, g2)
    wsem = (w0, w1, w2)

    n_chunk_rows = idx_hbm.shape[0]
    chunks_per_w = n_chunk_rows // NW
    ngroups = chunks_per_w // CPG
    wid = lax.axis_index("s") * NC + lax.axis_index("c")
    base_chunk = wid * chunks_per_w

    pltpu.sync_copy(idx_hbm.at[pl.ds(base_chunk, chunks_per_w)], idx_v)

    def fire_gathers(g, b):
        for h in range(CPG):
            pltpu.async_copy(
                table_hbm.at[idx_v.at[g * CPG + h]],
                rows[b].at[pl.ds(h * CHUNK, CHUNK)], gsem[b])

    def drain_gathers(g, b):
        for h in range(CPG):
            pltpu.make_async_copy(
                table_hbm.at[idx_v.at[g * CPG + h]],
                rows[b].at[pl.ds(h * CHUNK, CHUNK)], gsem[b]).wait()

    def fire_write(g, b):
        pltpu.async_copy(
            rows[b],
            out_hbm.at[pl.ds((base_chunk + g * CPG) * CHUNK, GROUP)],
            wsem[b])

    def drain_write(b):
        pltpu.make_async_copy(
            rows[b], out_hbm.at[pl.ds(0, GROUP)], wsem[b]).wait()

    fire_gathers(0, 0)
    fire_gathers(1, 1)
    drain_gathers(0, 0)
    fire_write(0, 0)
    fire_gathers(2, 2)
    drain_gathers(1, 1)
    fire_write(1, 1)

    @pl.loop(NSET, ngroups - NSET, step=NSET)
    def _(g0_):
        for h in range(NSET):
            g = g0_ + h
            b = h
            bp = (h + NSET - 1) % NSET
            drain_write(b)
            fire_gathers(g, b)
            drain_gathers(g - 1, bp)
            fire_write(g - 1, bp)

    glast = ngroups - 1
    drain_write(glast % NSET)
    fire_gathers(glast, glast % NSET)
    drain_gathers(glast - 1, (glast - 1) % NSET)
    fire_write(glast - 1, (glast - 1) % NSET)
    drain_gathers(glast, glast % NSET)
    fire_write(glast, glast % NSET)
    for b in range(NSET):
        drain_write((glast - 1 + b) % NSET)


def kernel(x, table):
    B, S = x.shape
    V, D = table.shape
    total = B * S
    idx2d = x.reshape(total // CHUNK, CHUNK)

    mesh = plsc.VectorSubcoreMesh(
        core_axis_name="c", subcore_axis_name="s",
        num_cores=NC, num_subcores=NS)

    run = pl.kernel(
        _gather_body,
        out_type=jax.ShapeDtypeStruct((total, D), jnp.float32),
        mesh=mesh,
        scratch_types=[
            pltpu.VMEM((total // CHUNK // NW, CHUNK), jnp.int32),
        ] + [pltpu.VMEM((GROUP, D), jnp.float32) for _ in range(NSET)]
          + [pltpu.SemaphoreType.DMA for _ in range(2 * NSET)],
    )
    out = run(table, idx2d)
    return out.reshape(B, S, D)
